# baseline (speedup 1.0000x reference)
import jax
import jax.numpy as jnp
from jax import lax
from jax.experimental import pallas as pl
from jax.experimental.pallas import tpu as pltpu
from jax.experimental.pallas import tpu_sc as plsc

N = 10000
E = 320000
H = 128

NC = 2
NS = 16
EPT = E // NS
C = 80
NCHUNK = EPT // C
PT = 640
N_PAD = NS * PT
HW = H + 16
BN_A = 2000
BN_B = 1280


def _tc_a_body(x_ref, w_ref, al_ref, ar_ref, h_ref, elr_ref):
    xb = x_ref[...]
    ones = jnp.ones((BN_A, 1), jnp.float32)
    zpad = jnp.zeros((BN_A, HW - H - 2), jnp.float32)
    cols = []
    for g in range(2):
        hg = jnp.dot(xb, w_ref[g], preferred_element_type=jnp.float32)
        el = jnp.sum(hg * al_ref[g][None, :], axis=-1)
        h_ref[g] = jnp.concatenate([hg, ones, el[:, None], zpad], axis=1)
        cols.append(el)
        cols.append(jnp.sum(hg * ar_ref[g][None, :], axis=-1))
    elr_ref[...] = jnp.stack(cols, axis=1)


def _tc_a(inp, W2, al2, ar2):
    return pl.pallas_call(
        _tc_a_body,
        grid=(N // BN_A,),
        in_specs=[
            pl.BlockSpec((BN_A, H), lambda i: (i, 0)),
            pl.BlockSpec((2, H, H), lambda i: (0, 0, 0)),
            pl.BlockSpec((2, H), lambda i: (0, 0)),
            pl.BlockSpec((2, H), lambda i: (0, 0)),
        ],
        out_specs=[
            pl.BlockSpec((2, BN_A, HW), lambda i: (0, i, 0)),
            pl.BlockSpec((BN_A, 4), lambda i: (i, 0)),
        ],
        out_shape=[
            jax.ShapeDtypeStruct((2, N, HW), jnp.float32),
            jax.ShapeDtypeStruct((N, 4), jnp.float32),
        ],
    )(inp, W2, al2, ar2)


def _sc_body(h2, elT, erT, srcO, dstO, acc_o,
             src_c0, dstg_c0, rows_v0, ela_v0, erb_v0,
             src_c1, dstg_c1, rows_v1, ela_v1, erb_v1,
             dst_s0, dst_s1, acc,
             sem_r0, sem_a0, sem_b0, sem_i0, sem_j0,
             sem_r1, sem_a1, sem_b1, sem_i1, sem_j1,
             sem_s0, sem_s1):
    c = lax.axis_index("c")
    s = lax.axis_index("s")

    zv = jnp.zeros((16,), jnp.float32)

    def _zrows(k, carry):
        for r in range(HW // 16):
            rows_v0[k, pl.ds(r * 16, 16)] = zv
        return carry
    lax.fori_loop(0, C, _zrows, 0)
    for m in range(PT // C):
        pltpu.sync_copy(rows_v0, acc.at[pl.ds(s * PT + m * C, C)])
    plsc.subcore_barrier()

    bufs = ((src_c0, dstg_c0, rows_v0, ela_v0, erb_v0,
             sem_r0, sem_a0, sem_b0, sem_i0, sem_j0, dst_s0, sem_s0),
            (src_c1, dstg_c1, rows_v1, ela_v1, erb_v1,
             sem_r1, sem_a1, sem_b1, sem_i1, sem_j1, dst_s1, sem_s1))
    gN = c * N

    def _fetch_idx(k, p):
        src_c, dstg_c = bufs[p][0:2]
        sem_i, sem_j = bufs[p][8:10]
        pltpu.async_copy(srcO.at[c, s, k], src_c, sem_i)
        pltpu.async_copy(dstO.at[c, s, k], dstg_c, sem_j)

    def _wait_idx(p):
        src_c, dstg_c = bufs[p][0:2]
        sem_i, sem_j = bufs[p][8:10]
        pltpu.make_async_copy(srcO.at[c, s, 0], src_c, sem_i).wait()
        pltpu.make_async_copy(dstO.at[c, s, 0], dstg_c, sem_j).wait()

    def _gather(p):
        src_c, dstg_c, rows_v, ela_v, erb_v = bufs[p][0:5]
        sem_r, sem_a, sem_b = bufs[p][5:8]
        pltpu.async_copy(h2.at[src_c], rows_v, sem_r)
        pltpu.async_copy(elT.at[src_c], ela_v, sem_a)
        pltpu.async_copy(erT.at[dstg_c], erb_v, sem_b)

    def _wait_gather(p):
        src_c, dstg_c, rows_v, ela_v, erb_v = bufs[p][0:5]
        sem_r, sem_a, sem_b = bufs[p][5:8]
        pltpu.make_async_copy(h2.at[src_c], rows_v, sem_r).wait()
        pltpu.make_async_copy(elT.at[src_c], ela_v, sem_a).wait()
        pltpu.make_async_copy(erT.at[dstg_c], erb_v, sem_b).wait()

    def _compute_scatter(p):
        _, dstg_c, rows_v, ela_v, erb_v = bufs[p][0:5]
        dst_s, sem_s = bufs[p][10:12]
        for j in range(C // 16):
            e = ela_v[pl.ds(j * 16, 16)] + erb_v[pl.ds(j * 16, 16)]
            e = jnp.where(e >= 0.0, e, e * 0.2)
            w = jnp.exp(e)
            for t in range(16):
                ws = w[t]
                jj = j * 16 + t
                for r in range(HW // 16):
                    sl = (jj, pl.ds(r * 16, 16))
                    rows_v[sl] = rows_v[sl] * ws
        for j in range(C // 16):
            dst_s[pl.ds(j * 16, 16)] = dstg_c[pl.ds(j * 16, 16)] - gN
        pltpu.async_copy(rows_v, acc.at[dst_s], sem_s, add=True)

    def _wait_scatter(p):
        rows_v = bufs[p][2]
        dst_s, sem_s = bufs[p][10:12]
        pltpu.make_async_copy(rows_v, acc.at[dst_s], sem_s).wait()

    _fetch_idx(jnp.int32(0), 0)
    _wait_idx(0)
    _gather(0)
    _fetch_idx(jnp.int32(1), 1)
    _wait_gather(0)
    _wait_idx(1)
    _gather(1)
    _compute_scatter(0)
    _fetch_idx(jnp.int32(2), 0)
    _wait_gather(1)
    _wait_idx(0)
    _wait_scatter(0)
    _gather(0)
    _compute_scatter(1)
    _fetch_idx(jnp.int32(3), 1)

    def _pair(g, carry):
        k0 = g * 2 + 2
        for p in (0, 1):
            k = k0 + p
            q = 1 - p
            _wait_gather(p)
            _wait_idx(q)
            _wait_scatter(q)
            _gather(q)
            _compute_scatter(p)
            _fetch_idx(k + 2, p)
        return carry
    lax.fori_loop(0, NCHUNK // 2 - 2, _pair, 0)

    _wait_gather(0)
    _wait_idx(1)
    _wait_scatter(1)
    _gather(1)
    _compute_scatter(0)
    _wait_gather(1)
    _wait_scatter(0)
    _compute_scatter(1)
    _wait_scatter(1)
    plsc.subcore_barrier()

    base_o = c * N_PAD + s * PT
    for m in range(PT // C):
        pltpu.sync_copy(acc.at[pl.ds(s * PT + m * C, C)],
                        acc_o.at[pl.ds(base_o + m * C, C)])


def _sc_call(h2f, elT, erT, srcO, dstO):
    mesh = plsc.VectorSubcoreMesh(core_axis_name="c", subcore_axis_name="s")
    f = pl.kernel(
        _sc_body,
        mesh=mesh,
        out_type=[
            jax.ShapeDtypeStruct((2 * N_PAD, HW), jnp.float32),
        ],
        scratch_types=(
            [pltpu.VMEM((C,), jnp.int32),
             pltpu.VMEM((C,), jnp.int32),
             pltpu.VMEM((C, HW), jnp.float32),
             pltpu.VMEM((C,), jnp.float32),
             pltpu.VMEM((C,), jnp.float32)] * 2
            + [pltpu.VMEM((C,), jnp.int32)] * 2
            + [pltpu.VMEM_SHARED((N_PAD, HW), jnp.float32)]
            + [pltpu.SemaphoreType.DMA] * 12
        ),
        compiler_params=pltpu.CompilerParams(use_tc_tiling_on_sc=False),
    )
    return f(h2f, elT, erT, srcO, dstO)


def _tc_b_body(az_ref, ah_ref, b_ref, o_ref):
    dz = az_ref[:, H:H + 1]
    dh = ah_ref[:, H:H + 1]
    dz = jnp.where(dz > 0.0, dz, 1.0)
    dh = jnp.where(dh > 0.0, dh, 1.0)
    z = jax.nn.sigmoid(az_ref[:, :H] / dz + b_ref[0][None, :])
    hc = jnp.tanh(ah_ref[:, :H] / dh + b_ref[1][None, :])
    o_ref[...] = (1.0 - z) * hc


def _tc_b(accs, b2):
    nblk = N_PAD // BN_B
    return pl.pallas_call(
        _tc_b_body,
        grid=(nblk,),
        in_specs=[
            pl.BlockSpec((BN_B, HW), lambda i: (i, 0)),
            pl.BlockSpec((BN_B, HW), lambda i: (nblk + i, 0)),
            pl.BlockSpec((2, H), lambda i: (0, 0)),
        ],
        out_specs=pl.BlockSpec((BN_B, H), lambda i: (i, 0)),
        out_shape=jax.ShapeDtypeStruct((N_PAD, H), jnp.float32),
    )(accs, accs, b2)


def kernel(x, edge_index, Ws, als, ars, bs):
    ei3 = edge_index.reshape(2, NS, NCHUNK, C)
    src3 = ei3[0]
    dst3 = ei3[1]
    srcO = jnp.stack([src3, src3 + N])
    dstO = jnp.stack([dst3, dst3 + N])
    out = x
    outs = []
    for i in range(Ws.shape[0]):
        W2 = Ws[i, 1:3, :H, :]
        al2 = als[i, 1:3]
        ar2 = ars[i, 1:3]
        b2 = bs[i, 1:3]
        h2, elr = _tc_a(out, W2, al2, ar2)
        elT = jnp.concatenate([elr[:, 0], elr[:, 2]])
        erT = jnp.concatenate([elr[:, 1], elr[:, 3]])
        (accs,) = _sc_call(h2.reshape(2 * N, HW), elT, erT, srcO, dstO)
        comb = _tc_b(accs, b2)
        out = comb[:N]
        outs.append(out)
    return jnp.stack(outs)


# speedup vs baseline: 1.0053x; 1.0053x over previous
import jax
import jax.numpy as jnp
from jax import lax
from jax.experimental import pallas as pl
from jax.experimental.pallas import tpu as pltpu
from jax.experimental.pallas import tpu_sc as plsc

N = 10000
E = 320000
H = 128

NC = 2
NS = 16
EPT = E // NS
C = 80
NCHUNK = EPT // C
PT = 640
N_PAD = NS * PT
HW = H + 16
BN_A = 2000
BN_B = 1280


def _tc_a_body(x_ref, w_ref, al_ref, ar_ref, h_ref, elr_ref):
    xb = x_ref[...]
    ones = jnp.ones((BN_A, 1), jnp.float32)
    zpad = jnp.zeros((BN_A, HW - H - 2), jnp.float32)
    cols = []
    for g in range(2):
        hg = jnp.dot(xb, w_ref[g], preferred_element_type=jnp.float32)
        el = jnp.sum(hg * al_ref[g][None, :], axis=-1)
        h_ref[g] = jnp.concatenate([hg, ones, el[:, None], zpad], axis=1)
        cols.append(el)
        cols.append(jnp.sum(hg * ar_ref[g][None, :], axis=-1))
    elr_ref[...] = jnp.stack(cols, axis=1)


def _tc_a(inp, W2, al2, ar2):
    return pl.pallas_call(
        _tc_a_body,
        grid=(N // BN_A,),
        in_specs=[
            pl.BlockSpec((BN_A, H), lambda i: (i, 0)),
            pl.BlockSpec((2, H, H), lambda i: (0, 0, 0)),
            pl.BlockSpec((2, H), lambda i: (0, 0)),
            pl.BlockSpec((2, H), lambda i: (0, 0)),
        ],
        out_specs=[
            pl.BlockSpec((2, BN_A, HW), lambda i: (0, i, 0)),
            pl.BlockSpec((BN_A, 4), lambda i: (i, 0)),
        ],
        out_shape=[
            jax.ShapeDtypeStruct((2, N, HW), jnp.float32),
            jax.ShapeDtypeStruct((N, 4), jnp.float32),
        ],
    )(inp, W2, al2, ar2)


def _sc_body(h2, elT, erT, srcO, dstO, acc_o,
             src_c0, dstg_c0, rows_v0, ela_v0, erb_v0,
             src_c1, dstg_c1, rows_v1, ela_v1, erb_v1,
             dst_s0, dst_s1, acc,
             sem_r0, sem_a0, sem_b0, sem_i0, sem_j0,
             sem_r1, sem_a1, sem_b1, sem_i1, sem_j1,
             sem_s0, sem_s1):
    c = lax.axis_index("c")
    s = lax.axis_index("s")

    zv = jnp.zeros((16,), jnp.float32)

    def _zrows(k, carry):
        for r in range(HW // 16):
            rows_v0[k, pl.ds(r * 16, 16)] = zv
        return carry
    lax.fori_loop(0, C, _zrows, 0)
    for m in range(PT // C):
        pltpu.sync_copy(rows_v0, acc.at[pl.ds(s * PT + m * C, C)])
    plsc.subcore_barrier()

    bufs = ((src_c0, dstg_c0, rows_v0, ela_v0, erb_v0,
             sem_r0, sem_a0, sem_b0, sem_i0, sem_j0, dst_s0, sem_s0),
            (src_c1, dstg_c1, rows_v1, ela_v1, erb_v1,
             sem_r1, sem_a1, sem_b1, sem_i1, sem_j1, dst_s1, sem_s1))
    gN = c * N

    def _fetch_idx(k, p):
        src_c, dstg_c = bufs[p][0:2]
        sem_i, sem_j = bufs[p][8:10]
        pltpu.async_copy(srcO.at[c, s, k], src_c, sem_i)
        pltpu.async_copy(dstO.at[c, s, k], dstg_c, sem_j)

    def _wait_idx(p):
        src_c, dstg_c = bufs[p][0:2]
        sem_i, sem_j = bufs[p][8:10]
        pltpu.make_async_copy(srcO.at[c, s, 0], src_c, sem_i).wait()
        pltpu.make_async_copy(dstO.at[c, s, 0], dstg_c, sem_j).wait()

    def _gather(p):
        src_c, dstg_c, rows_v, ela_v, erb_v = bufs[p][0:5]
        sem_r, sem_a, sem_b = bufs[p][5:8]
        pltpu.async_copy(h2.at[src_c], rows_v, sem_r)
        pltpu.async_copy(elT.at[src_c], ela_v, sem_a)
        pltpu.async_copy(erT.at[dstg_c], erb_v, sem_b)

    def _wait_gather(p):
        src_c, dstg_c, rows_v, ela_v, erb_v = bufs[p][0:5]
        sem_r, sem_a, sem_b = bufs[p][5:8]
        pltpu.make_async_copy(h2.at[src_c], rows_v, sem_r).wait()
        pltpu.make_async_copy(elT.at[src_c], ela_v, sem_a).wait()
        pltpu.make_async_copy(erT.at[dstg_c], erb_v, sem_b).wait()

    def _compute_scatter(p):
        _, dstg_c, rows_v, ela_v, erb_v = bufs[p][0:5]
        dst_s, sem_s = bufs[p][10:12]
        for j in range(C // 16):
            e = ela_v[pl.ds(j * 16, 16)] + erb_v[pl.ds(j * 16, 16)]
            e = jnp.where(e >= 0.0, e, e * 0.2)
            w = jnp.exp(e)
            for t in range(16):
                ws = w[t]
                jj = j * 16 + t
                for r in range(HW // 16):
                    sl = (jj, pl.ds(r * 16, 16))
                    rows_v[sl] = rows_v[sl] * ws
        for j in range(C // 16):
            dst_s[pl.ds(j * 16, 16)] = dstg_c[pl.ds(j * 16, 16)] - gN
        pltpu.async_copy(rows_v, acc.at[dst_s], sem_s, add=True)

    def _wait_scatter(p):
        rows_v = bufs[p][2]
        dst_s, sem_s = bufs[p][10:12]
        pltpu.make_async_copy(rows_v, acc.at[dst_s], sem_s).wait()

    _fetch_idx(jnp.int32(0), 0)
    _wait_idx(0)
    _gather(0)
    _fetch_idx(jnp.int32(1), 1)
    _wait_gather(0)
    _wait_idx(1)
    _gather(1)
    _compute_scatter(0)
    _fetch_idx(jnp.int32(2), 0)
    _wait_gather(1)
    _wait_idx(0)
    _wait_scatter(0)
    _gather(0)
    _compute_scatter(1)
    _fetch_idx(jnp.int32(3), 1)

    def _pair(g, carry):
        k0 = g * 2 + 2
        for p in (0, 1):
            k = k0 + p
            q = 1 - p
            _wait_gather(p)
            _wait_idx(q)
            _wait_scatter(q)
            _gather(q)
            _compute_scatter(p)
            _fetch_idx(k + 2, p)
        return carry
    lax.fori_loop(0, NCHUNK // 2 - 2, _pair, 0)

    _wait_gather(0)
    _wait_idx(1)
    _wait_scatter(1)
    _gather(1)
    _compute_scatter(0)
    _wait_gather(1)
    _wait_scatter(0)
    _compute_scatter(1)
    _wait_scatter(1)
    plsc.subcore_barrier()

    base_o = c * N_PAD + s * PT
    for m in range(PT // C):
        pltpu.sync_copy(acc.at[pl.ds(s * PT + m * C, C)],
                        acc_o.at[pl.ds(base_o + m * C, C)])


def _sc_call(h2f, elT, erT, srcO, dstO):
    mesh = plsc.VectorSubcoreMesh(core_axis_name="c", subcore_axis_name="s")
    f = pl.kernel(
        _sc_body,
        mesh=mesh,
        out_type=[
            jax.ShapeDtypeStruct((2 * N_PAD, HW), jnp.float32),
        ],
        scratch_types=(
            [pltpu.VMEM((C,), jnp.int32),
             pltpu.VMEM((C,), jnp.int32),
             pltpu.VMEM((C, HW), jnp.float32),
             pltpu.VMEM((C,), jnp.float32),
             pltpu.VMEM((C,), jnp.float32)] * 2
            + [pltpu.VMEM((C,), jnp.int32)] * 2
            + [pltpu.VMEM_SHARED((N_PAD, HW), jnp.float32)]
            + [pltpu.SemaphoreType.DMA] * 12
        ),
        compiler_params=pltpu.CompilerParams(use_tc_tiling_on_sc=False),
    )
    return f(h2f, elT, erT, srcO, dstO)


def _combine(az_ref, ah_ref, b_ref):
    dz = az_ref[:, H:H + 1]
    dh = ah_ref[:, H:H + 1]
    dz = jnp.where(dz > 0.0, dz, 1.0)
    dh = jnp.where(dh > 0.0, dh, 1.0)
    z = jax.nn.sigmoid(az_ref[:, :H] / dz + b_ref[0][None, :])
    hc = jnp.tanh(ah_ref[:, :H] / dh + b_ref[1][None, :])
    return (1.0 - z) * hc


def _tc_b_body(az_ref, ah_ref, b_ref, o_ref):
    o_ref[...] = _combine(az_ref, ah_ref, b_ref)


def _tc_b(accs, b2):
    nblk = N_PAD // BN_B
    return pl.pallas_call(
        _tc_b_body,
        grid=(nblk,),
        in_specs=[
            pl.BlockSpec((BN_B, HW), lambda i: (i, 0)),
            pl.BlockSpec((BN_B, HW), lambda i: (nblk + i, 0)),
            pl.BlockSpec((2, H), lambda i: (0, 0)),
        ],
        out_specs=pl.BlockSpec((BN_B, H), lambda i: (i, 0)),
        out_shape=jax.ShapeDtypeStruct((N_PAD, H), jnp.float32),
    )(accs, accs, b2)


def _tc_ab_body(az_ref, ah_ref, b_ref, w_ref, al_ref, ar_ref,
                o_ref, h_ref, elr_ref):
    comb = _combine(az_ref, ah_ref, b_ref)
    o_ref[...] = comb
    ones = jnp.ones((BN_B, 1), jnp.float32)
    zpad = jnp.zeros((BN_B, HW - H - 2), jnp.float32)
    cols = []
    for g in range(2):
        hg = jnp.dot(comb, w_ref[g], preferred_element_type=jnp.float32)
        el = jnp.sum(hg * al_ref[g][None, :], axis=-1)
        h_ref[g] = jnp.concatenate([hg, ones, el[:, None], zpad], axis=1)
        cols.append(el)
        cols.append(jnp.sum(hg * ar_ref[g][None, :], axis=-1))
    elr_ref[...] = jnp.stack(cols, axis=1)


def _tc_ab(accs, b2, W2n, al2n, ar2n):
    nblk = N_PAD // BN_B
    return pl.pallas_call(
        _tc_ab_body,
        grid=(nblk,),
        in_specs=[
            pl.BlockSpec((BN_B, HW), lambda i: (i, 0)),
            pl.BlockSpec((BN_B, HW), lambda i: (nblk + i, 0)),
            pl.BlockSpec((2, H), lambda i: (0, 0)),
            pl.BlockSpec((2, H, H), lambda i: (0, 0, 0)),
            pl.BlockSpec((2, H), lambda i: (0, 0)),
            pl.BlockSpec((2, H), lambda i: (0, 0)),
        ],
        out_specs=[
            pl.BlockSpec((BN_B, H), lambda i: (i, 0)),
            pl.BlockSpec((2, BN_B, HW), lambda i: (0, i, 0)),
            pl.BlockSpec((BN_B, 4), lambda i: (i, 0)),
        ],
        out_shape=[
            jax.ShapeDtypeStruct((N_PAD, H), jnp.float32),
            jax.ShapeDtypeStruct((2, N_PAD, HW), jnp.float32),
            jax.ShapeDtypeStruct((N_PAD, 4), jnp.float32),
        ],
    )(accs, accs, b2, W2n, al2n, ar2n)


def kernel(x, edge_index, Ws, als, ars, bs):
    ei3 = edge_index.reshape(2, NS, NCHUNK, C)
    src3 = ei3[0]
    dst3 = ei3[1]
    srcO = jnp.stack([src3, src3 + N])
    dstO = jnp.stack([dst3, dst3 + N])
    srcOp = jnp.stack([src3, src3 + N_PAD])
    dstOp = jnp.stack([dst3, dst3 + N_PAD])

    h2, elr = _tc_a(x, Ws[0, 1:3, :H, :], als[0, 1:3], ars[0, 1:3])
    elT = jnp.concatenate([elr[:, 0], elr[:, 2]])
    erT = jnp.concatenate([elr[:, 1], elr[:, 3]])
    (accs0,) = _sc_call(h2.reshape(2 * N, HW), elT, erT, srcO, dstO)

    out0, h2b, elrb = _tc_ab(accs0, bs[0, 1:3],
                             Ws[1, 1:3, :H, :], als[1, 1:3], ars[1, 1:3])
    elTp = jnp.concatenate([elrb[:, 0], elrb[:, 2]])
    erTp = jnp.concatenate([elrb[:, 1], elrb[:, 3]])
    (accs1,) = _sc_call(h2b.reshape(2 * N_PAD, HW), elTp, erTp, srcOp, dstOp)
    out1 = _tc_b(accs1, bs[1, 1:3])

    return jnp.stack([out0[:N], out1[:N]])


# speedup vs baseline: 1.0053x; 1.0001x over previous
import functools

import jax
import jax.numpy as jnp
from jax import lax
from jax.experimental import pallas as pl
from jax.experimental.pallas import tpu as pltpu
from jax.experimental.pallas import tpu_sc as plsc

N = 10000
E = 320000
H = 128

NC = 2
NS = 16
EPT = E // NS
C = 80
NCHUNK = EPT // C
PT = 640
N_PAD = NS * PT
HW = H + 16
BN_A = 2000
BN_B = 1280


def _tc_a_body(x_ref, w_ref, al_ref, ar_ref, h_ref, elr_ref):
    xb = x_ref[...]
    ones = jnp.ones((BN_A, 1), jnp.float32)
    zpad = jnp.zeros((BN_A, HW - H - 2), jnp.float32)
    cols = []
    for g in range(2):
        hg = jnp.dot(xb, w_ref[g], preferred_element_type=jnp.float32)
        el = jnp.sum(hg * al_ref[g][None, :], axis=-1)
        h_ref[g] = jnp.concatenate([hg, ones, el[:, None], zpad], axis=1)
        cols.append(el)
        cols.append(jnp.sum(hg * ar_ref[g][None, :], axis=-1))
    elr_ref[...] = jnp.stack(cols, axis=1)


def _tc_a(inp, W2, al2, ar2):
    return pl.pallas_call(
        _tc_a_body,
        grid=(N // BN_A,),
        in_specs=[
            pl.BlockSpec((BN_A, H), lambda i: (i, 0)),
            pl.BlockSpec((2, H, H), lambda i: (0, 0, 0)),
            pl.BlockSpec((2, H), lambda i: (0, 0)),
            pl.BlockSpec((2, H), lambda i: (0, 0)),
        ],
        out_specs=[
            pl.BlockSpec((2, BN_A, HW), lambda i: (0, i, 0)),
            pl.BlockSpec((BN_A, 4), lambda i: (i, 0)),
        ],
        out_shape=[
            jax.ShapeDtypeStruct((2, N, HW), jnp.float32),
            jax.ShapeDtypeStruct((N, 4), jnp.float32),
        ],
    )(inp, W2, al2, ar2)


def _sc_body(stride, h2, elT, erT, srcO, dstO, acc_o,
             src_c0, dstg_c0, rows_v0, ela_v0, erb_v0,
             src_c1, dstg_c1, rows_v1, ela_v1, erb_v1,
             dst_s0, dst_s1, acc,
             sem_r0, sem_a0, sem_b0, sem_i0, sem_j0,
             sem_r1, sem_a1, sem_b1, sem_i1, sem_j1,
             sem_s0, sem_s1):
    c = lax.axis_index("c")
    s = lax.axis_index("s")

    zv = jnp.zeros((16,), jnp.float32)

    def _zrows(k, carry):
        for r in range(HW // 16):
            rows_v0[k, pl.ds(r * 16, 16)] = zv
        return carry
    lax.fori_loop(0, C, _zrows, 0)
    for m in range(PT // C):
        pltpu.sync_copy(rows_v0, acc.at[pl.ds(s * PT + m * C, C)])
    plsc.subcore_barrier()

    bufs = ((src_c0, dstg_c0, rows_v0, ela_v0, erb_v0,
             sem_r0, sem_a0, sem_b0, sem_i0, sem_j0, dst_s0, sem_s0),
            (src_c1, dstg_c1, rows_v1, ela_v1, erb_v1,
             sem_r1, sem_a1, sem_b1, sem_i1, sem_j1, dst_s1, sem_s1))
    gN = c * stride

    def _fetch_idx(k, p):
        src_c, dstg_c = bufs[p][0:2]
        sem_i, sem_j = bufs[p][8:10]
        pltpu.async_copy(srcO.at[c, s, k], src_c, sem_i)
        pltpu.async_copy(dstO.at[c, s, k], dstg_c, sem_j)

    def _wait_idx(p):
        src_c, dstg_c = bufs[p][0:2]
        sem_i, sem_j = bufs[p][8:10]
        pltpu.make_async_copy(srcO.at[c, s, 0], src_c, sem_i).wait()
        pltpu.make_async_copy(dstO.at[c, s, 0], dstg_c, sem_j).wait()

    def _gather(p):
        src_c, dstg_c, rows_v, ela_v, erb_v = bufs[p][0:5]
        sem_r, sem_a, sem_b = bufs[p][5:8]
        pltpu.async_copy(h2.at[src_c], rows_v, sem_r)
        pltpu.async_copy(elT.at[src_c], ela_v, sem_a)
        pltpu.async_copy(erT.at[dstg_c], erb_v, sem_b)

    def _wait_gather(p):
        src_c, dstg_c, rows_v, ela_v, erb_v = bufs[p][0:5]
        sem_r, sem_a, sem_b = bufs[p][5:8]
        pltpu.make_async_copy(h2.at[src_c], rows_v, sem_r).wait()
        pltpu.make_async_copy(elT.at[src_c], ela_v, sem_a).wait()
        pltpu.make_async_copy(erT.at[dstg_c], erb_v, sem_b).wait()

    def _compute_scatter(p):
        _, dstg_c, rows_v, ela_v, erb_v = bufs[p][0:5]
        dst_s, sem_s = bufs[p][10:12]
        for j in range(C // 16):
            e = ela_v[pl.ds(j * 16, 16)] + erb_v[pl.ds(j * 16, 16)]
            e = jnp.where(e >= 0.0, e, e * 0.2)
            w = jnp.exp(e)
            for t in range(16):
                ws = w[t]
                jj = j * 16 + t
                for r in range(HW // 16):
                    sl = (jj, pl.ds(r * 16, 16))
                    rows_v[sl] = rows_v[sl] * ws
        for j in range(C // 16):
            dst_s[pl.ds(j * 16, 16)] = dstg_c[pl.ds(j * 16, 16)] - gN
        pltpu.async_copy(rows_v, acc.at[dst_s], sem_s, add=True)

    def _wait_scatter(p):
        rows_v = bufs[p][2]
        dst_s, sem_s = bufs[p][10:12]
        pltpu.make_async_copy(rows_v, acc.at[dst_s], sem_s).wait()

    _fetch_idx(jnp.int32(0), 0)
    _wait_idx(0)
    _gather(0)
    _fetch_idx(jnp.int32(1), 1)
    _wait_gather(0)
    _wait_idx(1)
    _gather(1)
    _compute_scatter(0)
    _fetch_idx(jnp.int32(2), 0)
    _wait_gather(1)
    _wait_idx(0)
    _wait_scatter(0)
    _gather(0)
    _compute_scatter(1)
    _fetch_idx(jnp.int32(3), 1)

    def _pair(g, carry):
        k0 = g * 2 + 2
        for p in (0, 1):
            k = k0 + p
            q = 1 - p
            _wait_gather(p)
            _wait_idx(q)
            _wait_scatter(q)
            _gather(q)
            _compute_scatter(p)
            _fetch_idx(k + 2, p)
        return carry
    lax.fori_loop(0, NCHUNK // 2 - 2, _pair, 0)

    _wait_gather(0)
    _wait_idx(1)
    _wait_scatter(1)
    _gather(1)
    _compute_scatter(0)
    _wait_gather(1)
    _wait_scatter(0)
    _compute_scatter(1)
    _wait_scatter(1)
    plsc.subcore_barrier()

    base_o = c * N_PAD + s * PT
    for m in range(PT // C):
        pltpu.sync_copy(acc.at[pl.ds(s * PT + m * C, C)],
                        acc_o.at[pl.ds(base_o + m * C, C)])


def _sc_call(h2f, elT, erT, srcO, dstO, stride):
    mesh = plsc.VectorSubcoreMesh(core_axis_name="c", subcore_axis_name="s")
    f = pl.kernel(
        functools.partial(_sc_body, stride),
        mesh=mesh,
        out_type=[
            jax.ShapeDtypeStruct((2 * N_PAD, HW), jnp.float32),
        ],
        scratch_types=(
            [pltpu.VMEM((C,), jnp.int32),
             pltpu.VMEM((C,), jnp.int32),
             pltpu.VMEM((C, HW), jnp.float32),
             pltpu.VMEM((C,), jnp.float32),
             pltpu.VMEM((C,), jnp.float32)] * 2
            + [pltpu.VMEM((C,), jnp.int32)] * 2
            + [pltpu.VMEM_SHARED((N_PAD, HW), jnp.float32)]
            + [pltpu.SemaphoreType.DMA] * 12
        ),
        compiler_params=pltpu.CompilerParams(use_tc_tiling_on_sc=False),
    )
    return f(h2f, elT, erT, srcO, dstO)


def _combine(az_ref, ah_ref, b_ref):
    dz = az_ref[:, H:H + 1]
    dh = ah_ref[:, H:H + 1]
    dz = jnp.where(dz > 0.0, dz, 1.0)
    dh = jnp.where(dh > 0.0, dh, 1.0)
    z = jax.nn.sigmoid(az_ref[:, :H] / dz + b_ref[0][None, :])
    hc = jnp.tanh(ah_ref[:, :H] / dh + b_ref[1][None, :])
    return (1.0 - z) * hc


def _tc_b_body(az_ref, ah_ref, b_ref, o_ref):
    o_ref[...] = _combine(az_ref, ah_ref, b_ref)


def _tc_b(accs, b2):
    nblk = N_PAD // BN_B
    return pl.pallas_call(
        _tc_b_body,
        grid=(nblk,),
        in_specs=[
            pl.BlockSpec((BN_B, HW), lambda i: (i, 0)),
            pl.BlockSpec((BN_B, HW), lambda i: (nblk + i, 0)),
            pl.BlockSpec((2, H), lambda i: (0, 0)),
        ],
        out_specs=pl.BlockSpec((BN_B, H), lambda i: (i, 0)),
        out_shape=jax.ShapeDtypeStruct((N_PAD, H), jnp.float32),
    )(accs, accs, b2)


def _tc_ab_body(az_ref, ah_ref, b_ref, w_ref, al_ref, ar_ref,
                o_ref, h_ref, elr_ref):
    comb = _combine(az_ref, ah_ref, b_ref)
    o_ref[...] = comb
    ones = jnp.ones((BN_B, 1), jnp.float32)
    zpad = jnp.zeros((BN_B, HW - H - 2), jnp.float32)
    cols = []
    for g in range(2):
        hg = jnp.dot(comb, w_ref[g], preferred_element_type=jnp.float32)
        el = jnp.sum(hg * al_ref[g][None, :], axis=-1)
        h_ref[g] = jnp.concatenate([hg, ones, el[:, None], zpad], axis=1)
        cols.append(el)
        cols.append(jnp.sum(hg * ar_ref[g][None, :], axis=-1))
    elr_ref[...] = jnp.stack(cols, axis=1)


def _tc_ab(accs, b2, W2n, al2n, ar2n):
    nblk = N_PAD // BN_B
    return pl.pallas_call(
        _tc_ab_body,
        grid=(nblk,),
        in_specs=[
            pl.BlockSpec((BN_B, HW), lambda i: (i, 0)),
            pl.BlockSpec((BN_B, HW), lambda i: (nblk + i, 0)),
            pl.BlockSpec((2, H), lambda i: (0, 0)),
            pl.BlockSpec((2, H, H), lambda i: (0, 0, 0)),
            pl.BlockSpec((2, H), lambda i: (0, 0)),
            pl.BlockSpec((2, H), lambda i: (0, 0)),
        ],
        out_specs=[
            pl.BlockSpec((BN_B, H), lambda i: (i, 0)),
            pl.BlockSpec((2, BN_B, HW), lambda i: (0, i, 0)),
            pl.BlockSpec((BN_B, 4), lambda i: (i, 0)),
        ],
        out_shape=[
            jax.ShapeDtypeStruct((N_PAD, H), jnp.float32),
            jax.ShapeDtypeStruct((2, N_PAD, HW), jnp.float32),
            jax.ShapeDtypeStruct((N_PAD, 4), jnp.float32),
        ],
    )(accs, accs, b2, W2n, al2n, ar2n)


def kernel(x, edge_index, Ws, als, ars, bs):
    ei3 = edge_index.reshape(2, NS, NCHUNK, C)
    src3 = ei3[0]
    dst3 = ei3[1]
    srcO = jnp.stack([src3, src3 + N])
    dstO = jnp.stack([dst3, dst3 + N])
    srcOp = jnp.stack([src3, src3 + N_PAD])
    dstOp = jnp.stack([dst3, dst3 + N_PAD])

    h2, elr = _tc_a(x, Ws[0, 1:3, :H, :], als[0, 1:3], ars[0, 1:3])
    elT = jnp.concatenate([elr[:, 0], elr[:, 2]])
    erT = jnp.concatenate([elr[:, 1], elr[:, 3]])
    (accs0,) = _sc_call(h2.reshape(2 * N, HW), elT, erT, srcO, dstO, N)

    out0, h2b, elrb = _tc_ab(accs0, bs[0, 1:3],
                             Ws[1, 1:3, :H, :], als[1, 1:3], ars[1, 1:3])
    elTp = jnp.concatenate([elrb[:, 0], elrb[:, 2]])
    erTp = jnp.concatenate([elrb[:, 1], elrb[:, 3]])
    (accs1,) = _sc_call(h2b.reshape(2 * N_PAD, HW), elTp, erTp, srcOp, dstOp, N_PAD)
    out1 = _tc_b(accs1, bs[1, 1:3])

    return jnp.stack([out0[:N], out1[:N]])
